# baseline (speedup 1.0000x reference)
import functools

import jax
import jax.numpy as jnp
from jax import lax
from jax.experimental import pallas as pl
from jax.experimental.pallas import tpu as pltpu
from jax.experimental.pallas import tpu_sc as plsc

_B = 4096
_L = 50
_EMBED_DIM = 64
_MASK_DIM = 16
_OUT_DIM = _EMBED_DIM + _MASK_DIM

_N = _B * _L
_NW = 32
_PER_W = _N // _NW
_C = 128
_CHUNKS = _PER_W // _C
_NBUF = 5
_D = 2
_T = _CHUNKS // _NBUF

_mesh = plsc.VectorSubcoreMesh(core_axis_name="c", subcore_axis_name="s")


@functools.partial(
    pl.kernel,
    mesh=_mesh,
    out_type=jax.ShapeDtypeStruct((_N, _OUT_DIM), jnp.float32),
    scratch_types=[
        pltpu.VMEM((_CHUNKS, _C), jnp.int32),
        pltpu.VMEM((_CHUNKS, _C), jnp.int32),
        pltpu.VMEM((2, _MASK_DIM), jnp.float32),
        pltpu.VMEM((_NBUF, _C, _EMBED_DIM), jnp.float32),
        pltpu.VMEM((_NBUF, _C, _OUT_DIM), jnp.float32),
    ]
    + [pltpu.SemaphoreType.DMA] * (2 * _NBUF),
    compiler_params=pltpu.CompilerParams(use_tc_tiling_on_sc=False),
)
def _embed_cat(sent_hbm, mask_hbm, word_hbm, mtab_hbm, out_hbm,
               sidx, midx, mtab, wbuf, cbuf, *sems):
    gsem = sems[:_NBUF]
    wsem = sems[_NBUF:]
    wid = lax.axis_index("s") * 2 + lax.axis_index("c")
    wbase = wid * _PER_W

    pltpu.sync_copy(sent_hbm.at[wid], sidx)
    pltpu.sync_copy(mask_hbm.at[wid], midx)
    pltpu.sync_copy(mtab_hbm, mtab)

    def fire_gather(c, s):
        pltpu.async_copy(word_hbm.at[sidx.at[c]], wbuf.at[s], gsem[s])

    def wait_gather(s):
        pltpu.make_async_copy(word_hbm.at[sidx.at[0]], wbuf.at[s],
                              gsem[s]).wait()

    def mask_fill(c, s):
        t0 = mtab[0, :]
        t1 = mtab[1, :]

        def body(v, carry):
            mv = midx[c, pl.ds(v * 16, 16)]
            base = v * 16
            for j in range(16):
                cbuf[s, base + j, pl.ds(_EMBED_DIM, _MASK_DIM)] = jnp.where(
                    mv[j] == 0, t0, t1)
            return carry

        lax.fori_loop(0, _C // 16, body, 0)

    def merge_words(s):
        def body(r, carry):
            for k in range(_EMBED_DIM // 16):
                cbuf[s, r, pl.ds(k * 16, 16)] = wbuf[s, r, pl.ds(k * 16, 16)]
            return carry

        lax.fori_loop(0, _C, body, 0)

    def fire_write(c, s):
        rows = pl.ds(wbase + c * _C, _C)
        pltpu.async_copy(cbuf.at[s], out_hbm.at[rows], wsem[s])

    def wait_write(s):
        rows = pl.ds(wbase, _C)
        pltpu.make_async_copy(cbuf.at[s], out_hbm.at[rows], wsem[s]).wait()

    for c in range(_D):
        fire_gather(c, c)

    def body(t, carry):
        for j in range(_NBUF):
            c = t * _NBUF + j
            sn = (j + _D) % _NBUF

            if j < _NBUF - _D:
                @pl.when(t > 0)
                def _():
                    wait_write(sn)

                fire_gather(c + _D, sn)
            else:
                wait_write(sn)

                @pl.when(t < _T - 1)
                def _():
                    fire_gather(c + _D, sn)

            mask_fill(c, j)
            wait_gather(j)
            merge_words(j)
            fire_write(c, j)
        return carry

    lax.fori_loop(0, _T, body, 0)

    for s in range(_D, _NBUF):
        wait_write(s)


def kernel(sent, mask, word_table, mask_table):
    s = sent.reshape(_NW, _CHUNKS, _C).astype(jnp.int32)
    m = mask.reshape(_NW, _CHUNKS, _C).astype(jnp.int32)
    out = _embed_cat(s, m, word_table, mask_table)
    return out.reshape(_B, _L, _OUT_DIM)


# speedup vs baseline: 1.0771x; 1.0771x over previous
import functools

import jax
import jax.numpy as jnp
from jax import lax
from jax.experimental import pallas as pl
from jax.experimental.pallas import tpu as pltpu
from jax.experimental.pallas import tpu_sc as plsc

_B = 4096
_L = 50
_EMBED_DIM = 64
_MASK_DIM = 16
_OUT_DIM = _EMBED_DIM + _MASK_DIM

_N = _B * _L
_NW = 32
_PER_W = _N // _NW
_C = 128
_CHUNKS = _PER_W // _C
_NBUF = 5
_D = 2
_T = _CHUNKS // _NBUF

_mesh = plsc.VectorSubcoreMesh(core_axis_name="c", subcore_axis_name="s")


@functools.partial(
    pl.kernel,
    mesh=_mesh,
    out_type=jax.ShapeDtypeStruct((_N, _OUT_DIM), jnp.float32),
    scratch_types=[
        pltpu.VMEM((_CHUNKS, _C), jnp.int32),
        pltpu.VMEM((_CHUNKS, _C), jnp.int32),
        pltpu.VMEM((2, _MASK_DIM), jnp.float32),
        pltpu.VMEM((_NBUF, _C, _EMBED_DIM), jnp.float32),
        pltpu.VMEM((_NBUF, _C, _MASK_DIM), jnp.float32),
    ]
    + [pltpu.SemaphoreType.DMA] * (3 * _NBUF),
    compiler_params=pltpu.CompilerParams(use_tc_tiling_on_sc=False),
)
def _embed_cat(sent_hbm, mask_hbm, word_hbm, mtab_hbm, out_hbm,
               sidx, midx, mtab, wbuf, mbuf, *sems):
    gsem = sems[:_NBUF]
    wsem = sems[_NBUF:2 * _NBUF]
    msem = sems[2 * _NBUF:]
    wid = lax.axis_index("s") * 2 + lax.axis_index("c")
    wbase = wid * _PER_W

    pltpu.sync_copy(sent_hbm.at[wid], sidx)
    pltpu.sync_copy(mask_hbm.at[wid], midx)
    pltpu.sync_copy(mtab_hbm, mtab)

    def fire_gather(c, s):
        pltpu.async_copy(word_hbm.at[sidx.at[c]], wbuf.at[s], gsem[s])

    def wait_gather(s):
        pltpu.make_async_copy(word_hbm.at[sidx.at[0]], wbuf.at[s],
                              gsem[s]).wait()

    def mask_fill(c, s):
        t0 = mtab[0, :]
        t1 = mtab[1, :]

        def body(v, carry):
            mv = midx[c, pl.ds(v * 16, 16)]
            base = v * 16
            for j in range(16):
                mbuf[s, base + j, :] = jnp.where(mv[j] == 0, t0, t1)
            return carry

        lax.fori_loop(0, _C // 16, body, 0)

    def fire_write(c, s):
        rows = pl.ds(wbase + c * _C, _C)
        pltpu.async_copy(wbuf.at[s], out_hbm.at[rows, pl.ds(0, _EMBED_DIM)],
                         wsem[s])
        pltpu.async_copy(mbuf.at[s],
                         out_hbm.at[rows, pl.ds(_EMBED_DIM, _MASK_DIM)],
                         msem[s])

    def wait_write(s):
        rows = pl.ds(wbase, _C)
        pltpu.make_async_copy(wbuf.at[s],
                              out_hbm.at[rows, pl.ds(0, _EMBED_DIM)],
                              wsem[s]).wait()
        pltpu.make_async_copy(mbuf.at[s],
                              out_hbm.at[rows, pl.ds(_EMBED_DIM, _MASK_DIM)],
                              msem[s]).wait()

    for c in range(_D):
        fire_gather(c, c)

    def body(t, carry):
        for j in range(_NBUF):
            c = t * _NBUF + j
            sn = (j + _D) % _NBUF

            if j < _NBUF - _D:
                @pl.when(t > 0)
                def _():
                    wait_write(sn)

                fire_gather(c + _D, sn)
            else:
                wait_write(sn)

                @pl.when(t < _T - 1)
                def _():
                    fire_gather(c + _D, sn)

            mask_fill(c, j)
            wait_gather(j)
            fire_write(c, j)
        return carry

    lax.fori_loop(0, _T, body, 0)

    for s in range(_D, _NBUF):
        wait_write(s)


def kernel(sent, mask, word_table, mask_table):
    s = sent.reshape(_NW, _CHUNKS, _C).astype(jnp.int32)
    m = mask.reshape(_NW, _CHUNKS, _C).astype(jnp.int32)
    out = _embed_cat(s, m, word_table, mask_table)
    return out.reshape(_B, _L, _OUT_DIM)
